# baseline (speedup 1.0000x reference)
import functools

import jax
import jax.numpy as jnp
from jax import lax
from jax.experimental import pallas as pl
from jax.experimental.pallas import tpu as pltpu
from jax.experimental.pallas import tpu_sc as plsc

N_NODES = 10000
N_PAD = 10240
E_EDGES = 320000
E_PAD = 327680
N_WORKERS = 32
CHUNKS = 160
CHUNK = 64
FEAT = 128
ROWS_PER_TILE = N_PAD // 16
BLK = 1024
GRID = N_PAD // BLK
G = 256
OUT = 12

@functools.cache
def _sc_mesh():
    return plsc.VectorSubcoreMesh(core_axis_name="c", subcore_axis_name="s")


def _zero_vmem(buf, nrows, ncols16):
    z = jnp.zeros((16,), jnp.float32)

    def body(i, _):
        for k in range(ncols16):
            buf[i, pl.ds(k * 16, 16)] = z
        return 0

    lax.fori_loop(0, nrows, body, 0)


def _zero_accum_slice(accum, zbuf, sid, zrows):
    base = sid * ROWS_PER_TILE

    def body(i, _):
        pltpu.sync_copy(zbuf, accum.at[pl.ds(base + i * zrows, zrows)])
        return 0

    lax.fori_loop(0, ROWS_PER_TILE // zrows, body, 0)


def _deg_body(dst_hbm, out_hbm, dst_c, ones_v, zbuf, accum):
    cid = lax.axis_index("c")
    sid = lax.axis_index("s")
    wid = cid * 16 + sid

    one = jnp.ones((16,), jnp.float32)

    def fill_ones(i, _):
        ones_v[i] = one
        return 0

    lax.fori_loop(0, CHUNK, fill_ones, 0)
    _zero_vmem(zbuf, 64, 1)
    _zero_accum_slice(accum, zbuf, sid, 64)
    plsc.subcore_barrier()

    def outer(t, _):
        pltpu.sync_copy(dst_hbm.at[wid, pl.ds(t * 8, 8)], dst_c)
        for p in range(8):
            pltpu.sync_copy(ones_v, accum.at[dst_c.at[p]], add=True)
        return 0

    lax.fori_loop(0, CHUNKS // 8, outer, 0)
    plsc.subcore_barrier()

    base = sid * ROWS_PER_TILE
    pltpu.sync_copy(
        accum.at[pl.ds(base, ROWS_PER_TILE)],
        out_hbm.at[cid, pl.ds(base, ROWS_PER_TILE)],
    )


def _edge_body(table_hbm, src_hbm, dst_hbm, out_hbm,
               src_c, dst_c, rows0, rows1, rows2, rows3, zbuf, accum,
               sem0, sem1, sem2, sem3):
    cid = lax.axis_index("c")
    sid = lax.axis_index("s")
    wid = cid * 16 + sid
    base = sid * ROWS_PER_TILE

    _zero_vmem(zbuf, 16, FEAT // 16)
    _zero_accum_slice(accum, zbuf, sid, 16)
    plsc.subcore_barrier()

    rows = (rows0, rows1, rows2, rows3)
    sems = (sem0, sem1, sem2, sem3)

    def outer(t, _):
        pltpu.sync_copy(src_hbm.at[wid, pl.ds(t * 16, 16)], src_c)
        pltpu.sync_copy(dst_hbm.at[wid, pl.ds(t * 16, 16)], dst_c)
        for q in range(4):
            gs = [
                pltpu.async_copy(
                    table_hbm.at[src_c.at[4 * q + p]], rows[p], sems[p])
                for p in range(4)
            ]
            for p in range(4):
                gs[p].wait()
                pltpu.sync_copy(
                    rows[p], accum.at[dst_c.at[4 * q + p]], add=True)
        return 0

    lax.fori_loop(0, CHUNKS // 16, outer, 0)
    plsc.subcore_barrier()

    pltpu.sync_copy(
        accum.at[pl.ds(base, ROWS_PER_TILE)],
        out_hbm.at[cid, pl.ds(base, ROWS_PER_TILE)],
    )


@functools.cache
def _deg_sc_kernel():
    return pl.kernel(
        _deg_body,
        out_type=jax.ShapeDtypeStruct((2, N_PAD, 16), jnp.float32),
        mesh=_sc_mesh(),
        scratch_types=[
            pltpu.VMEM((8, CHUNK), jnp.int32),
            pltpu.VMEM((CHUNK, 16), jnp.float32),
            pltpu.VMEM((64, 16), jnp.float32),
            pltpu.VMEM_SHARED((N_PAD, 16), jnp.float32),
        ],
    )


@functools.cache
def _edge_sc_kernel():
    return pl.kernel(
        _edge_body,
        out_type=jax.ShapeDtypeStruct((2, N_PAD, FEAT), jnp.float32),
        mesh=_sc_mesh(),
        scratch_types=[
            pltpu.VMEM((16, CHUNK), jnp.int32),
            pltpu.VMEM((16, CHUNK), jnp.int32),
            pltpu.VMEM((CHUNK, FEAT), jnp.float32),
            pltpu.VMEM((CHUNK, FEAT), jnp.float32),
            pltpu.VMEM((CHUNK, FEAT), jnp.float32),
            pltpu.VMEM((CHUNK, FEAT), jnp.float32),
            pltpu.VMEM((16, FEAT), jnp.float32),
            pltpu.VMEM_SHARED((N_PAD, FEAT), jnp.float32),
            pltpu.SemaphoreType.DMA,
            pltpu.SemaphoreType.DMA,
            pltpu.SemaphoreType.DMA,
            pltpu.SemaphoreType.DMA,
        ],
    )


def _dinv_block(cnt_blk):
    deg = cnt_blk[0, :, 0] + cnt_blk[1, :, 0] + 1.0
    return lax.rsqrt(deg)


def _row_mask(k):
    rows = k * BLK + lax.broadcasted_iota(jnp.int32, (BLK, 1), 0)
    return (rows < N_NODES).astype(jnp.float32)


def _tc_first(x_ref, w_ref, cnt_ref, h_ref, hn_ref):
    k = pl.program_id(0)
    h = jnp.dot(x_ref[...], w_ref[...], preferred_element_type=jnp.float32)
    dinv = _dinv_block(cnt_ref[...])
    h_ref[...] = h
    hn_ref[...] = h * dinv[:, None] * _row_mask(k)


def _tc_mid(s_ref, h_ref, cnt_ref, b_ref, w_ref, h2_ref, hn2_ref):
    k = pl.program_id(0)
    dinv = _dinv_block(cnt_ref[...])
    s = s_ref[0].astype(jnp.float32) + s_ref[1].astype(jnp.float32)
    out1 = jnp.maximum(
        dinv[:, None] * s + (dinv * dinv)[:, None] * h_ref[...] + b_ref[...], 0.0)
    h2 = jnp.dot(out1, w_ref[...], preferred_element_type=jnp.float32)
    h2_ref[...] = h2
    hn2_ref[...] = h2 * dinv[:, None] * _row_mask(k)


def _tc_last(s_ref, h_ref, cnt_ref, b_ref, batch_ref, desc_ref, wd_ref, bd_ref,
             wo_ref, bo_ref, out_ref, acc, cacc):
    k = pl.program_id(0)

    @pl.when(k == 0)
    def _():
        acc[...] = jnp.zeros_like(acc)
        cacc[...] = jnp.zeros_like(cacc)

    dinv = _dinv_block(cnt_ref[...])
    s = s_ref[0].astype(jnp.float32) + s_ref[1].astype(jnp.float32)
    out2 = jnp.maximum(
        dinv[:, None] * s + (dinv * dinv)[:, None] * h_ref[...] + b_ref[...], 0.0)
    b = batch_ref[0, 0, :]
    onehot = (b[None, :] == lax.broadcasted_iota(jnp.int32, (G, BLK), 0)
              ).astype(jnp.float32)
    acc[...] += jnp.dot(onehot, out2, preferred_element_type=jnp.float32)
    cacc[...] += jnp.broadcast_to(jnp.sum(onehot, axis=1)[:, None], (G, FEAT))

    @pl.when(k == GRID - 1)
    def _():
        pooled = acc[...] / jnp.maximum(cacc[...], 1.0)
        d = jnp.maximum(
            jnp.dot(desc_ref[...], wd_ref[...],
                    preferred_element_type=jnp.float32) + bd_ref[...], 0.0)
        cat = jnp.concatenate([pooled, d], axis=1)
        out_ref[...] = jnp.dot(cat, wo_ref[...],
                               preferred_element_type=jnp.float32) + bo_ref[...]


def kernel(x, edge_index, batch, descriptors, W1, b1, W2, b2, Wd, bd, Wo, bo):
    f32 = jnp.float32
    x_pad = jnp.pad(x, ((0, N_PAD - N_NODES), (0, 0)))
    src3 = jnp.pad(edge_index[0], (0, E_PAD - E_EDGES),
                   constant_values=N_NODES).reshape(N_WORKERS, CHUNKS, CHUNK)
    dst3 = jnp.pad(edge_index[1], (0, E_PAD - E_EDGES),
                   constant_values=N_NODES).reshape(N_WORKERS, CHUNKS, CHUNK)
    batch3 = jnp.pad(batch, (0, N_PAD - N_NODES),
                     constant_values=G).reshape(GRID, 1, BLK)
    b1r = b1.reshape(1, FEAT)
    b2r = b2.reshape(1, FEAT)
    bdr = bd.reshape(1, FEAT)
    bor = bo.reshape(1, OUT)

    cnt = _deg_sc_kernel()(dst3)

    row_blk = lambda k: (k, 0)
    cnt_spec = pl.BlockSpec((2, BLK, 16), lambda k: (0, k, 0))
    s_spec = pl.BlockSpec((2, BLK, FEAT), lambda k: (0, k, 0))
    full = lambda shape: pl.BlockSpec(shape, lambda k: tuple(0 for _ in shape))

    h1, hn1 = pl.pallas_call(
        _tc_first,
        grid=(GRID,),
        in_specs=[
            pl.BlockSpec((BLK, FEAT), row_blk),
            full((FEAT, FEAT)),
            cnt_spec,
        ],
        out_specs=[pl.BlockSpec((BLK, FEAT), row_blk)] * 2,
        out_shape=[jax.ShapeDtypeStruct((N_PAD, FEAT), f32),
                   jax.ShapeDtypeStruct((N_PAD, FEAT), f32)],
    )(x_pad, W1, cnt)

    s1 = _edge_sc_kernel()(hn1, src3, dst3)

    h2, hn2 = pl.pallas_call(
        _tc_mid,
        grid=(GRID,),
        in_specs=[
            s_spec,
            pl.BlockSpec((BLK, FEAT), row_blk),
            cnt_spec,
            full((1, FEAT)),
            full((FEAT, FEAT)),
        ],
        out_specs=[pl.BlockSpec((BLK, FEAT), row_blk)] * 2,
        out_shape=[jax.ShapeDtypeStruct((N_PAD, FEAT), f32),
                   jax.ShapeDtypeStruct((N_PAD, FEAT), f32)],
    )(s1, h1, cnt, b1r, W2)

    s2 = _edge_sc_kernel()(hn2, src3, dst3)

    out = pl.pallas_call(
        _tc_last,
        grid=(GRID,),
        in_specs=[
            s_spec,
            pl.BlockSpec((BLK, FEAT), row_blk),
            cnt_spec,
            full((1, FEAT)),
            pl.BlockSpec((1, 1, BLK), lambda k: (k, 0, 0)),
            full((G, 64)),
            full((64, FEAT)),
            full((1, FEAT)),
            full((2 * FEAT, OUT)),
            full((1, OUT)),
        ],
        out_specs=pl.BlockSpec((G, OUT), lambda k: (0, 0)),
        out_shape=jax.ShapeDtypeStruct((G, OUT), f32),
        scratch_shapes=[
            pltpu.VMEM((G, FEAT), f32),
            pltpu.VMEM((G, FEAT), f32),
        ],
    )(s2, h2, cnt, b2r, batch3, descriptors, Wd, bdr, Wo, bor)

    return out


# speedup vs baseline: 1.0780x; 1.0780x over previous
import functools

import jax
import jax.numpy as jnp
from jax import lax
from jax.experimental import pallas as pl
from jax.experimental.pallas import tpu as pltpu
from jax.experimental.pallas import tpu_sc as plsc

N_NODES = 10000
N_PAD = 10240
E_EDGES = 320000
E_PAD = 327680
N_WORKERS = 32
CHUNKS = 160
CHUNK = 64
FEAT = 128
ROWS_PER_TILE = N_PAD // 16
BLK = 1024
GRID = N_PAD // BLK
G = 256
OUT = 12

@functools.cache
def _sc_mesh():
    return plsc.VectorSubcoreMesh(core_axis_name="c", subcore_axis_name="s")


def _zero_vmem(buf, nrows, ncols16):
    z = jnp.zeros((16,), jnp.float32)

    def body(i, _):
        for k in range(ncols16):
            buf[i, pl.ds(k * 16, 16)] = z
        return 0

    lax.fori_loop(0, nrows, body, 0)


def _zero_accum_slice(accum, zbuf, sid, zrows):
    base = sid * ROWS_PER_TILE

    def body(i, _):
        pltpu.sync_copy(zbuf, accum.at[pl.ds(base + i * zrows, zrows)])
        return 0

    lax.fori_loop(0, ROWS_PER_TILE // zrows, body, 0)


def _deg_body(dst_hbm, out_hbm, dst_c, ones_v, zbuf, accum):
    cid = lax.axis_index("c")
    sid = lax.axis_index("s")
    wid = cid * 16 + sid

    one = jnp.ones((16,), jnp.float32)

    def fill_ones(i, _):
        ones_v[i] = one
        return 0

    lax.fori_loop(0, CHUNK, fill_ones, 0)
    _zero_vmem(zbuf, 64, 1)
    _zero_accum_slice(accum, zbuf, sid, 64)
    plsc.subcore_barrier()

    def outer(t, _):
        pltpu.sync_copy(dst_hbm.at[wid, pl.ds(t * 8, 8)], dst_c)
        for p in range(8):
            pltpu.sync_copy(ones_v, accum.at[dst_c.at[p]], add=True)
        return 0

    lax.fori_loop(0, CHUNKS // 8, outer, 0)
    plsc.subcore_barrier()

    base = sid * ROWS_PER_TILE
    pltpu.sync_copy(
        accum.at[pl.ds(base, ROWS_PER_TILE)],
        out_hbm.at[cid, pl.ds(base, ROWS_PER_TILE)],
    )


def _edge_body(table_hbm, src_hbm, dst_hbm, out_hbm,
               src_c, dst_c, rows0, rows1, rows2, rows3, zbuf, accum,
               sem0, sem1, sem2, sem3, ssem0, ssem1, ssem2, ssem3):
    cid = lax.axis_index("c")
    sid = lax.axis_index("s")
    wid = cid * 16 + sid
    base = sid * ROWS_PER_TILE

    _zero_vmem(zbuf, 16, FEAT // 16)
    _zero_accum_slice(accum, zbuf, sid, 16)
    plsc.subcore_barrier()

    rows = (rows0, rows1, rows2, rows3)
    gsems = (sem0, sem1, sem2, sem3)
    ssems = (ssem0, ssem1, ssem2, ssem3)

    def outer(t, _):
        pltpu.sync_copy(src_hbm.at[wid, pl.ds(t * 16, 16)], src_c)
        pltpu.sync_copy(dst_hbm.at[wid, pl.ds(t * 16, 16)], dst_c)
        gs = [
            pltpu.async_copy(table_hbm.at[src_c.at[p]], rows[p], gsems[p])
            for p in range(4)
        ]
        ss = [None] * 4
        for j in range(16):
            p = j % 4
            gs[p].wait()
            ss[p] = pltpu.async_copy(
                rows[p], accum.at[dst_c.at[j]], ssems[p], add=True)
            if j + 4 < 16:
                ss[p].wait()
                gs[p] = pltpu.async_copy(
                    table_hbm.at[src_c.at[j + 4]], rows[p], gsems[p])
        for p in range(4):
            ss[p].wait()
        return 0

    lax.fori_loop(0, CHUNKS // 16, outer, 0)
    plsc.subcore_barrier()

    pltpu.sync_copy(
        accum.at[pl.ds(base, ROWS_PER_TILE)],
        out_hbm.at[cid, pl.ds(base, ROWS_PER_TILE)],
    )


@functools.cache
def _deg_sc_kernel():
    return pl.kernel(
        _deg_body,
        out_type=jax.ShapeDtypeStruct((2, N_PAD, 16), jnp.float32),
        mesh=_sc_mesh(),
        scratch_types=[
            pltpu.VMEM((8, CHUNK), jnp.int32),
            pltpu.VMEM((CHUNK, 16), jnp.float32),
            pltpu.VMEM((64, 16), jnp.float32),
            pltpu.VMEM_SHARED((N_PAD, 16), jnp.float32),
        ],
    )


@functools.cache
def _edge_sc_kernel():
    return pl.kernel(
        _edge_body,
        out_type=jax.ShapeDtypeStruct((2, N_PAD, FEAT), jnp.float32),
        mesh=_sc_mesh(),
        scratch_types=[
            pltpu.VMEM((16, CHUNK), jnp.int32),
            pltpu.VMEM((16, CHUNK), jnp.int32),
            pltpu.VMEM((CHUNK, FEAT), jnp.float32),
            pltpu.VMEM((CHUNK, FEAT), jnp.float32),
            pltpu.VMEM((CHUNK, FEAT), jnp.float32),
            pltpu.VMEM((CHUNK, FEAT), jnp.float32),
            pltpu.VMEM((16, FEAT), jnp.float32),
            pltpu.VMEM_SHARED((N_PAD, FEAT), jnp.float32),
            pltpu.SemaphoreType.DMA,
            pltpu.SemaphoreType.DMA,
            pltpu.SemaphoreType.DMA,
            pltpu.SemaphoreType.DMA,
            pltpu.SemaphoreType.DMA,
            pltpu.SemaphoreType.DMA,
            pltpu.SemaphoreType.DMA,
            pltpu.SemaphoreType.DMA,
        ],
    )


def _dinv_block(cnt_blk):
    deg = cnt_blk[0, :, 0] + cnt_blk[1, :, 0] + 1.0
    return lax.rsqrt(deg)


def _row_mask(k):
    rows = k * BLK + lax.broadcasted_iota(jnp.int32, (BLK, 1), 0)
    return (rows < N_NODES).astype(jnp.float32)


def _tc_first(x_ref, w_ref, cnt_ref, h_ref, hn_ref):
    k = pl.program_id(0)
    h = jnp.dot(x_ref[...], w_ref[...], preferred_element_type=jnp.float32)
    dinv = _dinv_block(cnt_ref[...])
    h_ref[...] = h
    hn_ref[...] = h * dinv[:, None] * _row_mask(k)


def _tc_mid(s_ref, h_ref, cnt_ref, b_ref, w_ref, h2_ref, hn2_ref):
    k = pl.program_id(0)
    dinv = _dinv_block(cnt_ref[...])
    s = s_ref[0].astype(jnp.float32) + s_ref[1].astype(jnp.float32)
    out1 = jnp.maximum(
        dinv[:, None] * s + (dinv * dinv)[:, None] * h_ref[...] + b_ref[...], 0.0)
    h2 = jnp.dot(out1, w_ref[...], preferred_element_type=jnp.float32)
    h2_ref[...] = h2
    hn2_ref[...] = h2 * dinv[:, None] * _row_mask(k)


def _tc_last(s_ref, h_ref, cnt_ref, b_ref, batch_ref, desc_ref, wd_ref, bd_ref,
             wo_ref, bo_ref, out_ref, acc, cacc):
    k = pl.program_id(0)

    @pl.when(k == 0)
    def _():
        acc[...] = jnp.zeros_like(acc)
        cacc[...] = jnp.zeros_like(cacc)

    dinv = _dinv_block(cnt_ref[...])
    s = s_ref[0].astype(jnp.float32) + s_ref[1].astype(jnp.float32)
    out2 = jnp.maximum(
        dinv[:, None] * s + (dinv * dinv)[:, None] * h_ref[...] + b_ref[...], 0.0)
    b = batch_ref[0, 0, :]
    onehot = (b[None, :] == lax.broadcasted_iota(jnp.int32, (G, BLK), 0)
              ).astype(jnp.float32)
    acc[...] += jnp.dot(onehot, out2, preferred_element_type=jnp.float32)
    cacc[...] += jnp.broadcast_to(jnp.sum(onehot, axis=1)[:, None], (G, FEAT))

    @pl.when(k == GRID - 1)
    def _():
        pooled = acc[...] / jnp.maximum(cacc[...], 1.0)
        d = jnp.maximum(
            jnp.dot(desc_ref[...], wd_ref[...],
                    preferred_element_type=jnp.float32) + bd_ref[...], 0.0)
        cat = jnp.concatenate([pooled, d], axis=1)
        out_ref[...] = jnp.dot(cat, wo_ref[...],
                               preferred_element_type=jnp.float32) + bo_ref[...]


def kernel(x, edge_index, batch, descriptors, W1, b1, W2, b2, Wd, bd, Wo, bo):
    f32 = jnp.float32
    x_pad = jnp.pad(x, ((0, N_PAD - N_NODES), (0, 0)))
    src3 = jnp.pad(edge_index[0], (0, E_PAD - E_EDGES),
                   constant_values=N_NODES).reshape(N_WORKERS, CHUNKS, CHUNK)
    dst3 = jnp.pad(edge_index[1], (0, E_PAD - E_EDGES),
                   constant_values=N_NODES).reshape(N_WORKERS, CHUNKS, CHUNK)
    batch3 = jnp.pad(batch, (0, N_PAD - N_NODES),
                     constant_values=G).reshape(GRID, 1, BLK)
    b1r = b1.reshape(1, FEAT)
    b2r = b2.reshape(1, FEAT)
    bdr = bd.reshape(1, FEAT)
    bor = bo.reshape(1, OUT)

    cnt = _deg_sc_kernel()(dst3)

    row_blk = lambda k: (k, 0)
    cnt_spec = pl.BlockSpec((2, BLK, 16), lambda k: (0, k, 0))
    s_spec = pl.BlockSpec((2, BLK, FEAT), lambda k: (0, k, 0))
    full = lambda shape: pl.BlockSpec(shape, lambda k: tuple(0 for _ in shape))

    h1, hn1 = pl.pallas_call(
        _tc_first,
        grid=(GRID,),
        in_specs=[
            pl.BlockSpec((BLK, FEAT), row_blk),
            full((FEAT, FEAT)),
            cnt_spec,
        ],
        out_specs=[pl.BlockSpec((BLK, FEAT), row_blk)] * 2,
        out_shape=[jax.ShapeDtypeStruct((N_PAD, FEAT), f32),
                   jax.ShapeDtypeStruct((N_PAD, FEAT), f32)],
    )(x_pad, W1, cnt)

    s1 = _edge_sc_kernel()(hn1, src3, dst3)

    h2, hn2 = pl.pallas_call(
        _tc_mid,
        grid=(GRID,),
        in_specs=[
            s_spec,
            pl.BlockSpec((BLK, FEAT), row_blk),
            cnt_spec,
            full((1, FEAT)),
            full((FEAT, FEAT)),
        ],
        out_specs=[pl.BlockSpec((BLK, FEAT), row_blk)] * 2,
        out_shape=[jax.ShapeDtypeStruct((N_PAD, FEAT), f32),
                   jax.ShapeDtypeStruct((N_PAD, FEAT), f32)],
    )(s1, h1, cnt, b1r, W2)

    s2 = _edge_sc_kernel()(hn2, src3, dst3)

    out = pl.pallas_call(
        _tc_last,
        grid=(GRID,),
        in_specs=[
            s_spec,
            pl.BlockSpec((BLK, FEAT), row_blk),
            cnt_spec,
            full((1, FEAT)),
            pl.BlockSpec((1, 1, BLK), lambda k: (k, 0, 0)),
            full((G, 64)),
            full((64, FEAT)),
            full((1, FEAT)),
            full((2 * FEAT, OUT)),
            full((1, OUT)),
        ],
        out_specs=pl.BlockSpec((G, OUT), lambda k: (0, 0)),
        out_shape=jax.ShapeDtypeStruct((G, OUT), f32),
        scratch_shapes=[
            pltpu.VMEM((G, FEAT), f32),
            pltpu.VMEM((G, FEAT), f32),
        ],
    )(s2, h2, cnt, b2r, batch3, descriptors, Wd, bdr, Wo, bor)

    return out


# speedup vs baseline: 1.1005x; 1.0209x over previous
import functools

import jax
import jax.numpy as jnp
from jax import lax
from jax.experimental import pallas as pl
from jax.experimental.pallas import tpu as pltpu
from jax.experimental.pallas import tpu_sc as plsc

N_NODES = 10000
N_PAD = 10240
E_EDGES = 320000
E_PAD = 327680
N_WORKERS = 32
CHUNKS = 320
CHUNK = 32
NBUF = 8
BLKC = 16
FEAT = 128
ROWS_PER_TILE = N_PAD // 16
BLK = 1024
GRID = N_PAD // BLK
G = 256
OUT = 12

@functools.cache
def _sc_mesh():
    return plsc.VectorSubcoreMesh(core_axis_name="c", subcore_axis_name="s")


def _zero_vmem(buf, nrows, ncols16):
    z = jnp.zeros((16,), jnp.float32)

    def body(i, _):
        for k in range(ncols16):
            buf[i, pl.ds(k * 16, 16)] = z
        return 0

    lax.fori_loop(0, nrows, body, 0)


def _zero_accum_slice(accum, zbuf, sid, zrows):
    base = sid * ROWS_PER_TILE

    def body(i, _):
        pltpu.sync_copy(zbuf, accum.at[pl.ds(base + i * zrows, zrows)])
        return 0

    lax.fori_loop(0, ROWS_PER_TILE // zrows, body, 0)


def _deg_body(dst_hbm, out_hbm, dst_c, ones_v, zbuf, accum):
    cid = lax.axis_index("c")
    sid = lax.axis_index("s")
    wid = cid * 16 + sid

    one = jnp.ones((16,), jnp.float32)

    def fill_ones(i, _):
        ones_v[i] = one
        return 0

    lax.fori_loop(0, CHUNK, fill_ones, 0)
    _zero_vmem(zbuf, 64, 1)
    _zero_accum_slice(accum, zbuf, sid, 64)
    plsc.subcore_barrier()

    def outer(t, _):
        pltpu.sync_copy(dst_hbm.at[wid, pl.ds(t * 8, 8)], dst_c)
        for p in range(8):
            pltpu.sync_copy(ones_v, accum.at[dst_c.at[p]], add=True)
        return 0

    lax.fori_loop(0, CHUNKS // 8, outer, 0)
    plsc.subcore_barrier()

    base = sid * ROWS_PER_TILE
    pltpu.sync_copy(
        accum.at[pl.ds(base, ROWS_PER_TILE)],
        out_hbm.at[cid, pl.ds(base, ROWS_PER_TILE)],
    )


def _edge_body(table_hbm, src_hbm, dst_hbm, out_hbm, src_c, dst_c, *rest):
    rows = rest[:NBUF]
    zbuf = rest[NBUF]
    accum = rest[NBUF + 1]
    gsems = rest[NBUF + 2:2 * NBUF + 2]
    ssems = rest[2 * NBUF + 2:]
    cid = lax.axis_index("c")
    sid = lax.axis_index("s")
    wid = cid * 16 + sid
    base = sid * ROWS_PER_TILE

    _zero_vmem(zbuf, 16, FEAT // 16)
    _zero_accum_slice(accum, zbuf, sid, 16)
    plsc.subcore_barrier()

    def outer(t, _):
        pltpu.sync_copy(src_hbm.at[wid, pl.ds(t * BLKC, BLKC)], src_c)
        pltpu.sync_copy(dst_hbm.at[wid, pl.ds(t * BLKC, BLKC)], dst_c)
        gs = [
            pltpu.async_copy(table_hbm.at[src_c.at[p]], rows[p], gsems[p])
            for p in range(NBUF)
        ]
        ss = [None] * NBUF
        for j in range(BLKC):
            p = j % NBUF
            gs[p].wait()
            ss[p] = pltpu.async_copy(
                rows[p], accum.at[dst_c.at[j]], ssems[p], add=True)
            if j + NBUF < BLKC:
                ss[p].wait()
                gs[p] = pltpu.async_copy(
                    table_hbm.at[src_c.at[j + NBUF]], rows[p], gsems[p])
        for p in range(NBUF):
            ss[p].wait()
        return 0

    lax.fori_loop(0, CHUNKS // BLKC, outer, 0)
    plsc.subcore_barrier()

    pltpu.sync_copy(
        accum.at[pl.ds(base, ROWS_PER_TILE)],
        out_hbm.at[cid, pl.ds(base, ROWS_PER_TILE)],
    )


@functools.cache
def _deg_sc_kernel():
    return pl.kernel(
        _deg_body,
        out_type=jax.ShapeDtypeStruct((2, N_PAD, 16), jnp.float32),
        mesh=_sc_mesh(),
        scratch_types=[
            pltpu.VMEM((8, CHUNK), jnp.int32),
            pltpu.VMEM((CHUNK, 16), jnp.float32),
            pltpu.VMEM((64, 16), jnp.float32),
            pltpu.VMEM_SHARED((N_PAD, 16), jnp.float32),
        ],
    )


@functools.cache
def _edge_sc_kernel():
    return pl.kernel(
        _edge_body,
        out_type=jax.ShapeDtypeStruct((2, N_PAD, FEAT), jnp.float32),
        mesh=_sc_mesh(),
        scratch_types=(
            [pltpu.VMEM((BLKC, CHUNK), jnp.int32)] * 2
            + [pltpu.VMEM((CHUNK, FEAT), jnp.float32)] * NBUF
            + [pltpu.VMEM((16, FEAT), jnp.float32)]
            + [pltpu.VMEM_SHARED((N_PAD, FEAT), jnp.float32)]
            + [pltpu.SemaphoreType.DMA] * (2 * NBUF)
        ),
    )


def _dinv_block(cnt_blk):
    deg = cnt_blk[0, :, 0] + cnt_blk[1, :, 0] + 1.0
    return lax.rsqrt(deg)


def _row_mask(k):
    rows = k * BLK + lax.broadcasted_iota(jnp.int32, (BLK, 1), 0)
    return (rows < N_NODES).astype(jnp.float32)


def _tc_first(x_ref, w_ref, cnt_ref, h_ref, hn_ref):
    k = pl.program_id(0)
    h = jnp.dot(x_ref[...], w_ref[...], preferred_element_type=jnp.float32)
    dinv = _dinv_block(cnt_ref[...])
    h_ref[...] = h
    hn_ref[...] = h * dinv[:, None] * _row_mask(k)


def _tc_mid(s_ref, h_ref, cnt_ref, b_ref, w_ref, h2_ref, hn2_ref):
    k = pl.program_id(0)
    dinv = _dinv_block(cnt_ref[...])
    s = s_ref[0].astype(jnp.float32) + s_ref[1].astype(jnp.float32)
    out1 = jnp.maximum(
        dinv[:, None] * s + (dinv * dinv)[:, None] * h_ref[...] + b_ref[...], 0.0)
    h2 = jnp.dot(out1, w_ref[...], preferred_element_type=jnp.float32)
    h2_ref[...] = h2
    hn2_ref[...] = h2 * dinv[:, None] * _row_mask(k)


def _tc_last(s_ref, h_ref, cnt_ref, b_ref, batch_ref, desc_ref, wd_ref, bd_ref,
             wo_ref, bo_ref, out_ref, acc, cacc):
    k = pl.program_id(0)

    @pl.when(k == 0)
    def _():
        acc[...] = jnp.zeros_like(acc)
        cacc[...] = jnp.zeros_like(cacc)

    dinv = _dinv_block(cnt_ref[...])
    s = s_ref[0].astype(jnp.float32) + s_ref[1].astype(jnp.float32)
    out2 = jnp.maximum(
        dinv[:, None] * s + (dinv * dinv)[:, None] * h_ref[...] + b_ref[...], 0.0)
    b = batch_ref[0, 0, :]
    onehot = (b[None, :] == lax.broadcasted_iota(jnp.int32, (G, BLK), 0)
              ).astype(jnp.float32)
    acc[...] += jnp.dot(onehot, out2, preferred_element_type=jnp.float32)
    cacc[...] += jnp.broadcast_to(jnp.sum(onehot, axis=1)[:, None], (G, FEAT))

    @pl.when(k == GRID - 1)
    def _():
        pooled = acc[...] / jnp.maximum(cacc[...], 1.0)
        d = jnp.maximum(
            jnp.dot(desc_ref[...], wd_ref[...],
                    preferred_element_type=jnp.float32) + bd_ref[...], 0.0)
        cat = jnp.concatenate([pooled, d], axis=1)
        out_ref[...] = jnp.dot(cat, wo_ref[...],
                               preferred_element_type=jnp.float32) + bo_ref[...]


def kernel(x, edge_index, batch, descriptors, W1, b1, W2, b2, Wd, bd, Wo, bo):
    f32 = jnp.float32
    x_pad = jnp.pad(x, ((0, N_PAD - N_NODES), (0, 0)))
    src3 = jnp.pad(edge_index[0], (0, E_PAD - E_EDGES),
                   constant_values=N_NODES).reshape(N_WORKERS, CHUNKS, CHUNK)
    dst3 = jnp.pad(edge_index[1], (0, E_PAD - E_EDGES),
                   constant_values=N_NODES).reshape(N_WORKERS, CHUNKS, CHUNK)
    batch3 = jnp.pad(batch, (0, N_PAD - N_NODES),
                     constant_values=G).reshape(GRID, 1, BLK)
    b1r = b1.reshape(1, FEAT)
    b2r = b2.reshape(1, FEAT)
    bdr = bd.reshape(1, FEAT)
    bor = bo.reshape(1, OUT)

    cnt = _deg_sc_kernel()(dst3)

    row_blk = lambda k: (k, 0)
    cnt_spec = pl.BlockSpec((2, BLK, 16), lambda k: (0, k, 0))
    s_spec = pl.BlockSpec((2, BLK, FEAT), lambda k: (0, k, 0))
    full = lambda shape: pl.BlockSpec(shape, lambda k: tuple(0 for _ in shape))

    h1, hn1 = pl.pallas_call(
        _tc_first,
        grid=(GRID,),
        in_specs=[
            pl.BlockSpec((BLK, FEAT), row_blk),
            full((FEAT, FEAT)),
            cnt_spec,
        ],
        out_specs=[pl.BlockSpec((BLK, FEAT), row_blk)] * 2,
        out_shape=[jax.ShapeDtypeStruct((N_PAD, FEAT), f32),
                   jax.ShapeDtypeStruct((N_PAD, FEAT), f32)],
    )(x_pad, W1, cnt)

    s1 = _edge_sc_kernel()(hn1, src3, dst3)

    h2, hn2 = pl.pallas_call(
        _tc_mid,
        grid=(GRID,),
        in_specs=[
            s_spec,
            pl.BlockSpec((BLK, FEAT), row_blk),
            cnt_spec,
            full((1, FEAT)),
            full((FEAT, FEAT)),
        ],
        out_specs=[pl.BlockSpec((BLK, FEAT), row_blk)] * 2,
        out_shape=[jax.ShapeDtypeStruct((N_PAD, FEAT), f32),
                   jax.ShapeDtypeStruct((N_PAD, FEAT), f32)],
    )(s1, h1, cnt, b1r, W2)

    s2 = _edge_sc_kernel()(hn2, src3, dst3)

    out = pl.pallas_call(
        _tc_last,
        grid=(GRID,),
        in_specs=[
            s_spec,
            pl.BlockSpec((BLK, FEAT), row_blk),
            cnt_spec,
            full((1, FEAT)),
            pl.BlockSpec((1, 1, BLK), lambda k: (k, 0, 0)),
            full((G, 64)),
            full((64, FEAT)),
            full((1, FEAT)),
            full((2 * FEAT, OUT)),
            full((1, OUT)),
        ],
        out_specs=pl.BlockSpec((G, OUT), lambda k: (0, 0)),
        out_shape=jax.ShapeDtypeStruct((G, OUT), f32),
        scratch_shapes=[
            pltpu.VMEM((G, FEAT), f32),
            pltpu.VMEM((G, FEAT), f32),
        ],
    )(s2, h2, cnt, b2r, batch3, descriptors, Wd, bdr, Wo, bor)

    return out


# speedup vs baseline: 1.1327x; 1.0293x over previous
import functools

import jax
import jax.numpy as jnp
from jax import lax
from jax.experimental import pallas as pl
from jax.experimental.pallas import tpu as pltpu
from jax.experimental.pallas import tpu_sc as plsc

N_NODES = 10000
N_PAD = 10240
E_EDGES = 320000
E_PAD = 327680
N_WORKERS = 32
CHUNKS = 320
CHUNK = 32
NBUF = 8
BLKC = 16
FEAT = 128
ROWS_PER_TILE = N_PAD // 16
BLK = 1024
GRID = N_PAD // BLK
G = 256
OUT = 12

@functools.cache
def _sc_mesh():
    return plsc.VectorSubcoreMesh(core_axis_name="c", subcore_axis_name="s")


def _zero_vmem(buf, nrows, ncols16):
    z = jnp.zeros((16,), jnp.float32)

    def body(i, _):
        for k in range(ncols16):
            buf[i, pl.ds(k * 16, 16)] = z
        return 0

    lax.fori_loop(0, nrows, body, 0)


def _zero_accum_slice(accum, zbuf, sid, zrows, sem):
    base = sid * ROWS_PER_TILE
    cps = [
        pltpu.async_copy(zbuf, accum.at[pl.ds(base + i * zrows, zrows)], sem)
        for i in range(ROWS_PER_TILE // zrows)
    ]
    for c in cps:
        c.wait()


def _writeback_slice(accum, out_hbm, cid, sid, width, sem):
    base = sid * ROWS_PER_TILE
    q = ROWS_PER_TILE // 4
    cps = [
        pltpu.async_copy(
            accum.at[pl.ds(base + i * q, q)],
            out_hbm.at[cid, pl.ds(base + i * q, q)],
            sem,
        )
        for i in range(4)
    ]
    for c in cps:
        c.wait()


def _deg_body(dst_hbm, out_hbm, dst_c, ones_v, zbuf, accum, sem, wsem):
    cid = lax.axis_index("c")
    sid = lax.axis_index("s")
    wid = cid * 16 + sid

    one = jnp.ones((16,), jnp.float32)

    def fill_ones(i, _):
        ones_v[i] = one
        return 0

    lax.fori_loop(0, CHUNK, fill_ones, 0)
    _zero_vmem(zbuf, 64, 1)
    _zero_accum_slice(accum, zbuf, sid, 64, sem)
    plsc.subcore_barrier()

    def outer(t, _):
        pltpu.sync_copy(dst_hbm.at[wid, pl.ds(t * 16, 16)], dst_c)
        cps = [
            pltpu.async_copy(ones_v, accum.at[dst_c.at[p]], sem, add=True)
            for p in range(16)
        ]
        for c in cps:
            c.wait()
        return 0

    lax.fori_loop(0, CHUNKS // 16, outer, 0)
    plsc.subcore_barrier()
    _writeback_slice(accum, out_hbm, cid, sid, 16, wsem)


def _edge_body(table_hbm, src_hbm, dst_hbm, out_hbm, src_c, dst_c, *rest):
    rows = rest[:NBUF]
    zbuf = rest[NBUF]
    accum = rest[NBUF + 1]
    gsems = rest[NBUF + 2:2 * NBUF + 2]
    ssems = rest[2 * NBUF + 2:3 * NBUF + 2]
    wsem = rest[3 * NBUF + 2]
    cid = lax.axis_index("c")
    sid = lax.axis_index("s")
    wid = cid * 16 + sid
    base = sid * ROWS_PER_TILE

    _zero_vmem(zbuf, 64, FEAT // 16)
    _zero_accum_slice(accum, zbuf, sid, 64, wsem)
    plsc.subcore_barrier()

    def outer(t, _):
        pltpu.sync_copy(src_hbm.at[wid, pl.ds(t * BLKC, BLKC)], src_c)
        pltpu.sync_copy(dst_hbm.at[wid, pl.ds(t * BLKC, BLKC)], dst_c)
        gs = [
            pltpu.async_copy(table_hbm.at[src_c.at[p]], rows[p], gsems[p])
            for p in range(NBUF)
        ]
        ss = [None] * NBUF
        for j in range(BLKC):
            p = j % NBUF
            gs[p].wait()
            ss[p] = pltpu.async_copy(
                rows[p], accum.at[dst_c.at[j]], ssems[p], add=True)
            if j + NBUF < BLKC:
                ss[p].wait()
                gs[p] = pltpu.async_copy(
                    table_hbm.at[src_c.at[j + NBUF]], rows[p], gsems[p])
        for p in range(NBUF):
            ss[p].wait()
        return 0

    lax.fori_loop(0, CHUNKS // BLKC, outer, 0)
    plsc.subcore_barrier()
    _writeback_slice(accum, out_hbm, cid, sid, FEAT, wsem)


@functools.cache
def _deg_sc_kernel():
    return pl.kernel(
        _deg_body,
        out_type=jax.ShapeDtypeStruct((2, N_PAD, 16), jnp.float32),
        mesh=_sc_mesh(),
        scratch_types=[
            pltpu.VMEM((16, CHUNK), jnp.int32),
            pltpu.VMEM((CHUNK, 16), jnp.float32),
            pltpu.VMEM((64, 16), jnp.float32),
            pltpu.VMEM_SHARED((N_PAD, 16), jnp.float32),
            pltpu.SemaphoreType.DMA,
            pltpu.SemaphoreType.DMA,
        ],
    )


@functools.cache
def _edge_sc_kernel():
    return pl.kernel(
        _edge_body,
        out_type=jax.ShapeDtypeStruct((2, N_PAD, FEAT), jnp.float32),
        mesh=_sc_mesh(),
        scratch_types=(
            [pltpu.VMEM((BLKC, CHUNK), jnp.int32)] * 2
            + [pltpu.VMEM((CHUNK, FEAT), jnp.float32)] * NBUF
            + [pltpu.VMEM((64, FEAT), jnp.float32)]
            + [pltpu.VMEM_SHARED((N_PAD, FEAT), jnp.float32)]
            + [pltpu.SemaphoreType.DMA] * (2 * NBUF + 1)
        ),
    )


def _dinv_block(cnt_blk):
    deg = cnt_blk[0, :, 0] + cnt_blk[1, :, 0] + 1.0
    return lax.rsqrt(deg)


def _row_mask(k):
    rows = k * BLK + lax.broadcasted_iota(jnp.int32, (BLK, 1), 0)
    return (rows < N_NODES).astype(jnp.float32)


def _tc_first(x_ref, w_ref, cnt_ref, h_ref, hn_ref):
    k = pl.program_id(0)
    h = jnp.dot(x_ref[...], w_ref[...], preferred_element_type=jnp.float32)
    dinv = _dinv_block(cnt_ref[...])
    h_ref[...] = h
    hn_ref[...] = h * dinv[:, None] * _row_mask(k)


def _tc_mid(s_ref, h_ref, cnt_ref, b_ref, w_ref, h2_ref, hn2_ref):
    k = pl.program_id(0)
    dinv = _dinv_block(cnt_ref[...])
    s = s_ref[0].astype(jnp.float32) + s_ref[1].astype(jnp.float32)
    out1 = jnp.maximum(
        dinv[:, None] * s + (dinv * dinv)[:, None] * h_ref[...] + b_ref[...], 0.0)
    h2 = jnp.dot(out1, w_ref[...], preferred_element_type=jnp.float32)
    h2_ref[...] = h2
    hn2_ref[...] = h2 * dinv[:, None] * _row_mask(k)


def _tc_last(s_ref, h_ref, cnt_ref, b_ref, batch_ref, desc_ref, wd_ref, bd_ref,
             wo_ref, bo_ref, out_ref, acc, cacc):
    k = pl.program_id(0)

    @pl.when(k == 0)
    def _():
        acc[...] = jnp.zeros_like(acc)
        cacc[...] = jnp.zeros_like(cacc)

    dinv = _dinv_block(cnt_ref[...])
    s = s_ref[0].astype(jnp.float32) + s_ref[1].astype(jnp.float32)
    out2 = jnp.maximum(
        dinv[:, None] * s + (dinv * dinv)[:, None] * h_ref[...] + b_ref[...], 0.0)
    b = batch_ref[0, 0, :]
    onehot = (b[None, :] == lax.broadcasted_iota(jnp.int32, (G, BLK), 0)
              ).astype(jnp.float32)
    acc[...] += jnp.dot(onehot, out2, preferred_element_type=jnp.float32)
    cacc[...] += jnp.broadcast_to(jnp.sum(onehot, axis=1)[:, None], (G, FEAT))

    @pl.when(k == GRID - 1)
    def _():
        pooled = acc[...] / jnp.maximum(cacc[...], 1.0)
        d = jnp.maximum(
            jnp.dot(desc_ref[...], wd_ref[...],
                    preferred_element_type=jnp.float32) + bd_ref[...], 0.0)
        cat = jnp.concatenate([pooled, d], axis=1)
        out_ref[...] = jnp.dot(cat, wo_ref[...],
                               preferred_element_type=jnp.float32) + bo_ref[...]


def kernel(x, edge_index, batch, descriptors, W1, b1, W2, b2, Wd, bd, Wo, bo):
    f32 = jnp.float32
    x_pad = jnp.pad(x, ((0, N_PAD - N_NODES), (0, 0)))
    src3 = jnp.pad(edge_index[0], (0, E_PAD - E_EDGES),
                   constant_values=N_NODES).reshape(N_WORKERS, CHUNKS, CHUNK)
    dst3 = jnp.pad(edge_index[1], (0, E_PAD - E_EDGES),
                   constant_values=N_NODES).reshape(N_WORKERS, CHUNKS, CHUNK)
    batch3 = jnp.pad(batch, (0, N_PAD - N_NODES),
                     constant_values=G).reshape(GRID, 1, BLK)
    b1r = b1.reshape(1, FEAT)
    b2r = b2.reshape(1, FEAT)
    bdr = bd.reshape(1, FEAT)
    bor = bo.reshape(1, OUT)

    cnt = _deg_sc_kernel()(dst3)

    row_blk = lambda k: (k, 0)
    cnt_spec = pl.BlockSpec((2, BLK, 16), lambda k: (0, k, 0))
    s_spec = pl.BlockSpec((2, BLK, FEAT), lambda k: (0, k, 0))
    full = lambda shape: pl.BlockSpec(shape, lambda k: tuple(0 for _ in shape))

    h1, hn1 = pl.pallas_call(
        _tc_first,
        grid=(GRID,),
        in_specs=[
            pl.BlockSpec((BLK, FEAT), row_blk),
            full((FEAT, FEAT)),
            cnt_spec,
        ],
        out_specs=[pl.BlockSpec((BLK, FEAT), row_blk)] * 2,
        out_shape=[jax.ShapeDtypeStruct((N_PAD, FEAT), f32),
                   jax.ShapeDtypeStruct((N_PAD, FEAT), f32)],
    )(x_pad, W1, cnt)

    s1 = _edge_sc_kernel()(hn1, src3, dst3)

    h2, hn2 = pl.pallas_call(
        _tc_mid,
        grid=(GRID,),
        in_specs=[
            s_spec,
            pl.BlockSpec((BLK, FEAT), row_blk),
            cnt_spec,
            full((1, FEAT)),
            full((FEAT, FEAT)),
        ],
        out_specs=[pl.BlockSpec((BLK, FEAT), row_blk)] * 2,
        out_shape=[jax.ShapeDtypeStruct((N_PAD, FEAT), f32),
                   jax.ShapeDtypeStruct((N_PAD, FEAT), f32)],
    )(s1, h1, cnt, b1r, W2)

    s2 = _edge_sc_kernel()(hn2, src3, dst3)

    out = pl.pallas_call(
        _tc_last,
        grid=(GRID,),
        in_specs=[
            s_spec,
            pl.BlockSpec((BLK, FEAT), row_blk),
            cnt_spec,
            full((1, FEAT)),
            pl.BlockSpec((1, 1, BLK), lambda k: (k, 0, 0)),
            full((G, 64)),
            full((64, FEAT)),
            full((1, FEAT)),
            full((2 * FEAT, OUT)),
            full((1, OUT)),
        ],
        out_specs=pl.BlockSpec((G, OUT), lambda k: (0, 0)),
        out_shape=jax.ShapeDtypeStruct((G, OUT), f32),
        scratch_shapes=[
            pltpu.VMEM((G, FEAT), f32),
            pltpu.VMEM((G, FEAT), f32),
        ],
    )(s2, h2, cnt, b2r, batch3, descriptors, Wd, bdr, Wo, bor)

    return out


# speedup vs baseline: 3.0815x; 2.7203x over previous
import functools

import jax
import jax.numpy as jnp
from jax import lax
from jax.experimental import pallas as pl
from jax.experimental.pallas import tpu as pltpu
from jax.experimental.pallas import tpu_sc as plsc

N_NODES = 10000
N_PAD = 10240
E_EDGES = 320000
E_PAD = 327680
N_WORKERS = 32
CHUNKS = 320
CHUNK = 32
NBUF = 8
BLKC = 16
FEAT = 128
ROWS_PER_TILE = N_PAD // 16
BLK = 1024
GRID = N_PAD // BLK
G = 256
OUT = 12

@functools.cache
def _sc_mesh():
    return plsc.VectorSubcoreMesh(core_axis_name="c", subcore_axis_name="s")


def _zero_vmem(buf, nrows, ncols16):
    z = jnp.zeros((16,), jnp.float32)

    def body(i, _):
        for k in range(ncols16):
            buf[i, pl.ds(k * 16, 16)] = z
        return 0

    lax.fori_loop(0, nrows, body, 0)


def _zero_accum_slice(accum, zbuf, sid, zrows, sem):
    base = sid * ROWS_PER_TILE
    cps = [
        pltpu.async_copy(zbuf, accum.at[pl.ds(base + i * zrows, zrows)], sem)
        for i in range(ROWS_PER_TILE // zrows)
    ]
    for c in cps:
        c.wait()


def _writeback_slice(accum, out_hbm, cid, sid, width, sem):
    base = sid * ROWS_PER_TILE
    q = ROWS_PER_TILE // 4
    cps = [
        pltpu.async_copy(
            accum.at[pl.ds(base + i * q, q)],
            out_hbm.at[cid, pl.ds(base + i * q, q)],
            sem,
        )
        for i in range(4)
    ]
    for c in cps:
        c.wait()


def _deg_body(dst_hbm, out_hbm, dst_c, ones_v, zbuf, accum, sem, wsem):
    cid = lax.axis_index("c")
    sid = lax.axis_index("s")
    wid = cid * 16 + sid

    one = jnp.ones((16,), jnp.float32)

    def fill_ones(i, _):
        ones_v[i] = one
        return 0

    lax.fori_loop(0, CHUNK, fill_ones, 0)
    _zero_vmem(zbuf, 64, 1)
    _zero_accum_slice(accum, zbuf, sid, 64, sem)
    plsc.subcore_barrier()

    def outer(t, _):
        pltpu.sync_copy(dst_hbm.at[wid, pl.ds(t * 16, 16)], dst_c)
        cps = [
            pltpu.async_copy(ones_v, accum.at[dst_c.at[p]], sem, add=True)
            for p in range(16)
        ]
        for c in cps:
            c.wait()
        return 0

    lax.fori_loop(0, CHUNKS // 16, outer, 0)
    plsc.subcore_barrier()
    _writeback_slice(accum, out_hbm, cid, sid, 16, wsem)


def _edge_body(table_hbm, src_hbm, dst_hbm, out_hbm, src_c, dst_c, *rest):
    rows = rest[:NBUF]
    zbuf = rest[NBUF]
    accum = rest[NBUF + 1]
    gsems = rest[NBUF + 2:2 * NBUF + 2]
    ssems = rest[2 * NBUF + 2:3 * NBUF + 2]
    wsem = rest[3 * NBUF + 2]
    cid = lax.axis_index("c")
    sid = lax.axis_index("s")
    wid = cid * 16 + sid
    base = sid * ROWS_PER_TILE

    _zero_vmem(zbuf, 64, FEAT // 16)
    _zero_accum_slice(accum, zbuf, sid, 64, wsem)
    plsc.subcore_barrier()

    def outer(t, _):
        pltpu.sync_copy(src_hbm.at[wid, pl.ds(t * BLKC, BLKC)], src_c)
        pltpu.sync_copy(dst_hbm.at[wid, pl.ds(t * BLKC, BLKC)], dst_c)
        gs = [
            pltpu.async_copy(table_hbm.at[src_c.at[p]], rows[p], gsems[p])
            for p in range(NBUF)
        ]
        ss = [None] * NBUF
        for j in range(BLKC):
            p = j % NBUF
            gs[p].wait()
            ss[p] = pltpu.async_copy(
                rows[p], accum.at[dst_c.at[j]], ssems[p], add=True)
            if j + NBUF < BLKC:
                ss[p].wait()
                gs[p] = pltpu.async_copy(
                    table_hbm.at[src_c.at[j + NBUF]], rows[p], gsems[p])
        for p in range(NBUF):
            ss[p].wait()
        return 0

    lax.fori_loop(0, CHUNKS // BLKC, outer, 0)
    plsc.subcore_barrier()
    _writeback_slice(accum, out_hbm, cid, sid, FEAT, wsem)


@functools.cache
def _deg_sc_kernel():
    return pl.kernel(
        _deg_body,
        out_type=jax.ShapeDtypeStruct((2, N_PAD, 16), jnp.float32),
        mesh=_sc_mesh(),
        scratch_types=[
            pltpu.VMEM((16, CHUNK), jnp.int32),
            pltpu.VMEM((CHUNK, 16), jnp.float32),
            pltpu.VMEM((64, 16), jnp.float32),
            pltpu.VMEM_SHARED((N_PAD, 16), jnp.float32),
            pltpu.SemaphoreType.DMA,
            pltpu.SemaphoreType.DMA,
        ],
    )


@functools.cache
def _edge_sc_kernel():
    return pl.kernel(
        _edge_body,
        out_type=jax.ShapeDtypeStruct((2, N_PAD, FEAT), jnp.float32),
        mesh=_sc_mesh(),
        scratch_types=(
            [pltpu.VMEM((BLKC, CHUNK), jnp.int32)] * 2
            + [pltpu.VMEM((CHUNK, FEAT), jnp.float32)] * NBUF
            + [pltpu.VMEM((64, FEAT), jnp.float32)]
            + [pltpu.VMEM_SHARED((N_PAD, FEAT), jnp.float32)]
            + [pltpu.SemaphoreType.DMA] * (2 * NBUF + 1)
        ),
    )


def _dinv_block(cnt_blk):
    deg = cnt_blk[0, :, 0] + cnt_blk[1, :, 0] + 1.0
    return lax.rsqrt(deg)


def _row_mask(k):
    rows = k * BLK + lax.broadcasted_iota(jnp.int32, (BLK, 1), 0)
    return (rows < N_NODES).astype(jnp.float32)


def _tc_first(x_ref, w_ref, cnt_ref, h_ref, hn_ref):
    k = pl.program_id(0)
    h = jnp.dot(x_ref[...], w_ref[...], preferred_element_type=jnp.float32)
    dinv = _dinv_block(cnt_ref[...])
    h_ref[...] = h
    hn_ref[...] = h * dinv[:, None] * _row_mask(k)


def _tc_mid(s_ref, h_ref, cnt_ref, b_ref, w_ref, h2_ref, hn2_ref):
    k = pl.program_id(0)
    dinv = _dinv_block(cnt_ref[...])
    s = s_ref[0].astype(jnp.float32) + s_ref[1].astype(jnp.float32)
    out1 = jnp.maximum(
        dinv[:, None] * s + (dinv * dinv)[:, None] * h_ref[...] + b_ref[...], 0.0)
    h2 = jnp.dot(out1, w_ref[...], preferred_element_type=jnp.float32)
    h2_ref[...] = h2
    hn2_ref[...] = h2 * dinv[:, None] * _row_mask(k)


def _tc_last(s_ref, h_ref, cnt_ref, b_ref, batch_ref, desc_ref, wd_ref, bd_ref,
             wo_ref, bo_ref, out_ref, acc, cacc):
    k = pl.program_id(0)

    @pl.when(k == 0)
    def _():
        acc[...] = jnp.zeros_like(acc)
        cacc[...] = jnp.zeros_like(cacc)

    dinv = _dinv_block(cnt_ref[...])
    s = s_ref[0].astype(jnp.float32) + s_ref[1].astype(jnp.float32)
    out2 = jnp.maximum(
        dinv[:, None] * s + (dinv * dinv)[:, None] * h_ref[...] + b_ref[...], 0.0)
    b = batch_ref[0, 0, :]
    onehot = (b[None, :] == lax.broadcasted_iota(jnp.int32, (G, BLK), 0)
              ).astype(jnp.float32)
    acc[...] += jnp.dot(onehot, out2, preferred_element_type=jnp.float32)
    cacc[...] += jnp.broadcast_to(jnp.sum(onehot, axis=1)[:, None], (G, FEAT))

    @pl.when(k == GRID - 1)
    def _():
        pooled = acc[...] / jnp.maximum(cacc[...], 1.0)
        d = jnp.maximum(
            jnp.dot(desc_ref[...], wd_ref[...],
                    preferred_element_type=jnp.float32) + bd_ref[...], 0.0)
        cat = jnp.concatenate([pooled, d], axis=1)
        out_ref[...] = jnp.dot(cat, wo_ref[...],
                               preferred_element_type=jnp.float32) + bo_ref[...]


def kernel(x, edge_index, batch, descriptors, W1, b1, W2, b2, Wd, bd, Wo, bo):
    f32 = jnp.float32
    x_pad = jnp.pad(x, ((0, N_PAD - N_NODES), (0, 0)))
    pad_i = jnp.arange(E_PAD - E_EDGES, dtype=jnp.int32)
    src_fill = (pad_i * 131) % N_NODES
    dst_fill = N_NODES + pad_i % (N_PAD - N_NODES)
    src3 = jnp.concatenate([edge_index[0], src_fill]
                           ).reshape(N_WORKERS, CHUNKS, CHUNK)
    dst3 = jnp.concatenate([edge_index[1], dst_fill]
                           ).reshape(N_WORKERS, CHUNKS, CHUNK)
    batch3 = jnp.pad(batch, (0, N_PAD - N_NODES),
                     constant_values=G).reshape(GRID, 1, BLK)
    b1r = b1.reshape(1, FEAT)
    b2r = b2.reshape(1, FEAT)
    bdr = bd.reshape(1, FEAT)
    bor = bo.reshape(1, OUT)

    cnt = _deg_sc_kernel()(dst3)

    row_blk = lambda k: (k, 0)
    cnt_spec = pl.BlockSpec((2, BLK, 16), lambda k: (0, k, 0))
    s_spec = pl.BlockSpec((2, BLK, FEAT), lambda k: (0, k, 0))
    full = lambda shape: pl.BlockSpec(shape, lambda k: tuple(0 for _ in shape))

    h1, hn1 = pl.pallas_call(
        _tc_first,
        grid=(GRID,),
        in_specs=[
            pl.BlockSpec((BLK, FEAT), row_blk),
            full((FEAT, FEAT)),
            cnt_spec,
        ],
        out_specs=[pl.BlockSpec((BLK, FEAT), row_blk)] * 2,
        out_shape=[jax.ShapeDtypeStruct((N_PAD, FEAT), f32),
                   jax.ShapeDtypeStruct((N_PAD, FEAT), f32)],
    )(x_pad, W1, cnt)

    s1 = _edge_sc_kernel()(hn1, src3, dst3)

    h2, hn2 = pl.pallas_call(
        _tc_mid,
        grid=(GRID,),
        in_specs=[
            s_spec,
            pl.BlockSpec((BLK, FEAT), row_blk),
            cnt_spec,
            full((1, FEAT)),
            full((FEAT, FEAT)),
        ],
        out_specs=[pl.BlockSpec((BLK, FEAT), row_blk)] * 2,
        out_shape=[jax.ShapeDtypeStruct((N_PAD, FEAT), f32),
                   jax.ShapeDtypeStruct((N_PAD, FEAT), f32)],
    )(s1, h1, cnt, b1r, W2)

    s2 = _edge_sc_kernel()(hn2, src3, dst3)

    out = pl.pallas_call(
        _tc_last,
        grid=(GRID,),
        in_specs=[
            s_spec,
            pl.BlockSpec((BLK, FEAT), row_blk),
            cnt_spec,
            full((1, FEAT)),
            pl.BlockSpec((1, 1, BLK), lambda k: (k, 0, 0)),
            full((G, 64)),
            full((64, FEAT)),
            full((1, FEAT)),
            full((2 * FEAT, OUT)),
            full((1, OUT)),
        ],
        out_specs=pl.BlockSpec((G, OUT), lambda k: (0, 0)),
        out_shape=jax.ShapeDtypeStruct((G, OUT), f32),
        scratch_shapes=[
            pltpu.VMEM((G, FEAT), f32),
            pltpu.VMEM((G, FEAT), f32),
        ],
    )(s2, h2, cnt, b2r, batch3, descriptors, Wd, bdr, Wo, bor)

    return out


# speedup vs baseline: 3.0830x; 1.0005x over previous
import functools

import jax
import jax.numpy as jnp
from jax import lax
from jax.experimental import pallas as pl
from jax.experimental.pallas import tpu as pltpu
from jax.experimental.pallas import tpu_sc as plsc

N_NODES = 10000
N_PAD = 10240
E_EDGES = 320000
E_PAD = 327680
N_WORKERS = 32
CHUNKS = 320
CHUNK = 32
NBUF = 8
BLKC = 16
FEAT = 128
ROWS_PER_TILE = N_PAD // 16
BLK = 1024
GRID = N_PAD // BLK
G = 256
OUT = 12

@functools.cache
def _sc_mesh():
    return plsc.VectorSubcoreMesh(core_axis_name="c", subcore_axis_name="s")


def _zero_vmem(buf, nrows, ncols16):
    z = jnp.zeros((16,), jnp.float32)

    def body(i, _):
        for k in range(ncols16):
            buf[i, pl.ds(k * 16, 16)] = z
        return 0

    lax.fori_loop(0, nrows, body, 0)


def _zero_accum_slice(accum, zbuf, sid, zrows, sem):
    base = sid * ROWS_PER_TILE
    cps = [
        pltpu.async_copy(zbuf, accum.at[pl.ds(base + i * zrows, zrows)], sem)
        for i in range(ROWS_PER_TILE // zrows)
    ]
    for c in cps:
        c.wait()


def _writeback_slice(accum, out_hbm, cid, sid, width, sem):
    base = sid * ROWS_PER_TILE
    q = ROWS_PER_TILE // 4
    cps = [
        pltpu.async_copy(
            accum.at[pl.ds(base + i * q, q)],
            out_hbm.at[cid, pl.ds(base + i * q, q)],
            sem,
        )
        for i in range(4)
    ]
    for c in cps:
        c.wait()


def _deg_body(dst_hbm, out_hbm, dst_c, ones_v, zbuf, accum, sem, wsem):
    cid = lax.axis_index("c")
    sid = lax.axis_index("s")
    wid = cid * 16 + sid

    one = jnp.ones((16,), jnp.float32)

    def fill_ones(i, _):
        ones_v[i] = one
        return 0

    lax.fori_loop(0, CHUNK, fill_ones, 0)
    _zero_vmem(zbuf, 64, 1)
    _zero_accum_slice(accum, zbuf, sid, 64, sem)
    plsc.subcore_barrier()

    def outer(t, _):
        pltpu.sync_copy(dst_hbm.at[wid, pl.ds(t * 16, 16)], dst_c)
        cps = [
            pltpu.async_copy(ones_v, accum.at[dst_c.at[p]], sem, add=True)
            for p in range(16)
        ]
        for c in cps:
            c.wait()
        return 0

    lax.fori_loop(0, CHUNKS // 16, outer, 0)
    plsc.subcore_barrier()
    _writeback_slice(accum, out_hbm, cid, sid, 16, wsem)


def _edge_body(table_hbm, src_hbm, dst_hbm, out_hbm, src_c, dst_c, *rest):
    rows = rest[:NBUF]
    zbuf = rest[NBUF]
    accum = rest[NBUF + 1]
    gsems = rest[NBUF + 2:2 * NBUF + 2]
    ssems = rest[2 * NBUF + 2:3 * NBUF + 2]
    wsem = rest[3 * NBUF + 2]
    cid = lax.axis_index("c")
    sid = lax.axis_index("s")
    wid = cid * 16 + sid
    base = sid * ROWS_PER_TILE

    _zero_vmem(zbuf, 64, FEAT // 16)
    _zero_accum_slice(accum, zbuf, sid, 64, wsem)
    plsc.subcore_barrier()

    def outer(t, _):
        pltpu.sync_copy(src_hbm.at[wid, pl.ds(t * BLKC, BLKC)], src_c)
        pltpu.sync_copy(dst_hbm.at[wid, pl.ds(t * BLKC, BLKC)], dst_c)
        gs = [
            pltpu.async_copy(table_hbm.at[src_c.at[p]], rows[p], gsems[p])
            for p in range(NBUF)
        ]
        ss = [None] * NBUF
        for j in range(BLKC):
            p = j % NBUF
            gs[p].wait()
            ss[p] = pltpu.async_copy(
                rows[p], accum.at[dst_c.at[j]], ssems[p], add=True)
            if j + NBUF < BLKC:
                ss[p].wait()
                gs[p] = pltpu.async_copy(
                    table_hbm.at[src_c.at[j + NBUF]], rows[p], gsems[p])
        for p in range(NBUF):
            ss[p].wait()
        return 0

    lax.fori_loop(0, CHUNKS // BLKC, outer, 0)
    plsc.subcore_barrier()
    _writeback_slice(accum, out_hbm, cid, sid, FEAT, wsem)


@functools.cache
def _deg_sc_kernel():
    return pl.kernel(
        _deg_body,
        out_type=jax.ShapeDtypeStruct((2, N_PAD, 16), jnp.float32),
        mesh=_sc_mesh(),
        scratch_types=[
            pltpu.VMEM((16, CHUNK), jnp.int32),
            pltpu.VMEM((CHUNK, 16), jnp.float32),
            pltpu.VMEM((64, 16), jnp.float32),
            pltpu.VMEM_SHARED((N_PAD, 16), jnp.float32),
            pltpu.SemaphoreType.DMA,
            pltpu.SemaphoreType.DMA,
        ],
    )


@functools.cache
def _edge_sc_kernel():
    return pl.kernel(
        _edge_body,
        out_type=jax.ShapeDtypeStruct((2, N_PAD, FEAT), jnp.float32),
        mesh=_sc_mesh(),
        scratch_types=(
            [pltpu.VMEM((BLKC, CHUNK), jnp.int32)] * 2
            + [pltpu.VMEM((CHUNK, FEAT), jnp.float32)] * NBUF
            + [pltpu.VMEM((64, FEAT), jnp.float32)]
            + [pltpu.VMEM_SHARED((N_PAD, FEAT), jnp.float32)]
            + [pltpu.SemaphoreType.DMA] * (2 * NBUF + 1)
        ),
    )


def _dinv_block(cnt_blk):
    deg = cnt_blk[0, :, 0] + cnt_blk[1, :, 0] + 1.0
    return lax.rsqrt(deg)


def _row_mask(k):
    rows = k * BLK + lax.broadcasted_iota(jnp.int32, (BLK, 1), 0)
    return (rows < N_NODES).astype(jnp.float32)


def _tc_first(x_ref, w_ref, cnt_ref, h_ref, hn_ref):
    k = pl.program_id(0)
    h = jnp.dot(x_ref[...], w_ref[...], preferred_element_type=jnp.float32)
    dinv = _dinv_block(cnt_ref[...])
    h_ref[...] = h
    hn_ref[...] = h * dinv[:, None] * _row_mask(k)


def _tc_mid(s_ref, h_ref, cnt_ref, b_ref, w_ref, h2_ref, hn2_ref):
    k = pl.program_id(0)
    dinv = _dinv_block(cnt_ref[...])
    s = s_ref[0].astype(jnp.float32) + s_ref[1].astype(jnp.float32)
    out1 = jnp.maximum(
        dinv[:, None] * s + (dinv * dinv)[:, None] * h_ref[...] + b_ref[...], 0.0)
    h2 = jnp.dot(out1, w_ref[...], preferred_element_type=jnp.float32)
    h2_ref[...] = h2
    hn2_ref[...] = h2 * dinv[:, None] * _row_mask(k)


def _tc_last(s_ref, h_ref, cnt_ref, b_ref, batch_ref, desc_ref, wd_ref, bd_ref,
             wo_ref, bo_ref, out_ref, acc, cacc):
    k = pl.program_id(0)

    @pl.when(k == 0)
    def _():
        acc[...] = jnp.zeros_like(acc)
        cacc[...] = jnp.zeros_like(cacc)

    dinv = _dinv_block(cnt_ref[...])
    s = s_ref[0].astype(jnp.float32) + s_ref[1].astype(jnp.float32)
    out2 = jnp.maximum(
        dinv[:, None] * s + (dinv * dinv)[:, None] * h_ref[...] + b_ref[...], 0.0)
    b = batch_ref[0, 0, :]
    onehot = (b[None, :] == lax.broadcasted_iota(jnp.int32, (G, BLK), 0)
              ).astype(jnp.float32)
    acc[...] += jnp.dot(onehot, out2, preferred_element_type=jnp.float32)
    cacc[...] += jnp.broadcast_to(jnp.sum(onehot, axis=1)[:, None], (G, FEAT))

    @pl.when(k == GRID - 1)
    def _():
        pooled = acc[...] / jnp.maximum(cacc[...], 1.0)
        d = jnp.maximum(
            jnp.dot(desc_ref[...], wd_ref[...],
                    preferred_element_type=jnp.float32) + bd_ref[...], 0.0)
        cat = jnp.concatenate([pooled, d], axis=1)
        out_ref[...] = jnp.dot(cat, wo_ref[...],
                               preferred_element_type=jnp.float32) + bo_ref[...]


def kernel(x, edge_index, batch, descriptors, W1, b1, W2, b2, Wd, bd, Wo, bo):
    f32 = jnp.float32
    x_pad = jnp.pad(x, ((0, N_PAD - N_NODES), (0, 0)))
    pad_i = jnp.arange(E_PAD - E_EDGES, dtype=jnp.int32)
    src_fill = N_NODES + pad_i % (N_PAD - N_NODES)
    dst_fill = N_NODES + pad_i % (N_PAD - N_NODES)
    src3 = jnp.concatenate([edge_index[0], src_fill]
                           ).reshape(N_WORKERS, CHUNKS, CHUNK)
    dst3 = jnp.concatenate([edge_index[1], dst_fill]
                           ).reshape(N_WORKERS, CHUNKS, CHUNK)
    batch3 = jnp.pad(batch, (0, N_PAD - N_NODES),
                     constant_values=G).reshape(GRID, 1, BLK)
    b1r = b1.reshape(1, FEAT)
    b2r = b2.reshape(1, FEAT)
    bdr = bd.reshape(1, FEAT)
    bor = bo.reshape(1, OUT)

    cnt = _deg_sc_kernel()(dst3)

    row_blk = lambda k: (k, 0)
    cnt_spec = pl.BlockSpec((2, BLK, 16), lambda k: (0, k, 0))
    s_spec = pl.BlockSpec((2, BLK, FEAT), lambda k: (0, k, 0))
    full = lambda shape: pl.BlockSpec(shape, lambda k: tuple(0 for _ in shape))

    h1, hn1 = pl.pallas_call(
        _tc_first,
        grid=(GRID,),
        in_specs=[
            pl.BlockSpec((BLK, FEAT), row_blk),
            full((FEAT, FEAT)),
            cnt_spec,
        ],
        out_specs=[pl.BlockSpec((BLK, FEAT), row_blk)] * 2,
        out_shape=[jax.ShapeDtypeStruct((N_PAD, FEAT), f32),
                   jax.ShapeDtypeStruct((N_PAD, FEAT), f32)],
    )(x_pad, W1, cnt)

    s1 = _edge_sc_kernel()(hn1, src3, dst3)

    h2, hn2 = pl.pallas_call(
        _tc_mid,
        grid=(GRID,),
        in_specs=[
            s_spec,
            pl.BlockSpec((BLK, FEAT), row_blk),
            cnt_spec,
            full((1, FEAT)),
            full((FEAT, FEAT)),
        ],
        out_specs=[pl.BlockSpec((BLK, FEAT), row_blk)] * 2,
        out_shape=[jax.ShapeDtypeStruct((N_PAD, FEAT), f32),
                   jax.ShapeDtypeStruct((N_PAD, FEAT), f32)],
    )(s1, h1, cnt, b1r, W2)

    s2 = _edge_sc_kernel()(hn2, src3, dst3)

    out = pl.pallas_call(
        _tc_last,
        grid=(GRID,),
        in_specs=[
            s_spec,
            pl.BlockSpec((BLK, FEAT), row_blk),
            cnt_spec,
            full((1, FEAT)),
            pl.BlockSpec((1, 1, BLK), lambda k: (k, 0, 0)),
            full((G, 64)),
            full((64, FEAT)),
            full((1, FEAT)),
            full((2 * FEAT, OUT)),
            full((1, OUT)),
        ],
        out_specs=pl.BlockSpec((G, OUT), lambda k: (0, 0)),
        out_shape=jax.ShapeDtypeStruct((G, OUT), f32),
        scratch_shapes=[
            pltpu.VMEM((G, FEAT), f32),
            pltpu.VMEM((G, FEAT), f32),
        ],
    )(s2, h2, cnt, b2r, batch3, descriptors, Wd, bdr, Wo, bor)

    return out
